# baseline (speedup 1.0000x reference)
import functools

import jax
import jax.numpy as jnp
from jax import lax
from jax.experimental import pallas as pl
from jax.experimental.pallas import tpu as pltpu
from jax.experimental.pallas import tpu_sc as plsc

_NC = 2
_NS = 16
_NW = _NC * _NS
_LANES = 16
_ICHUNK = 128


def _sc_mean_pool(idsT, table, batch, ctx, dim):
    bpw = batch // _NW
    inv = 1.0 / ctx
    nk = dim // _LANES
    unroll = 5

    mesh = plsc.VectorSubcoreMesh(core_axis_name="c", subcore_axis_name="s",
                                  num_cores=_NC, num_subcores=_NS)

    @functools.partial(
        pl.kernel,
        out_type=jax.ShapeDtypeStruct((batch, dim), jnp.float32),
        mesh=mesh,
        scratch_types=[
            pltpu.VMEM((ctx, bpw), jnp.int32),
            pltpu.VMEM((ctx * bpw, dim), jnp.float32),
            pltpu.VMEM((bpw, dim), jnp.float32),
            pltpu.SemaphoreType.DMA,
        ],
        compiler_params=pltpu.CompilerParams(use_tc_tiling_on_sc=False),
    )
    def body(idsT_hbm, table_hbm, out_hbm, idxT_v, rows_v, hid_v, sem):
        wid = lax.axis_index("s") * _NC + lax.axis_index("c")
        base = wid * bpw
        pltpu.sync_copy(idsT_hbm.at[:, pl.ds(base, bpw)], idxT_v)
        copies = [
            pltpu.async_copy(
                table_hbm.at[idxT_v.at[j]],
                rows_v.at[pl.ds(j * bpw, bpw)],
                sem,
            )
            for j in range(ctx)
        ]
        for cp in copies:
            cp.wait()

        def rbody(r, _):
            def jbody(j, accs):
                for u in range(unroll):
                    row = (j * unroll + u) * bpw
                    for k in range(nk):
                        accs = (accs[:k]
                                + (accs[k] + rows_v[row + r,
                                                    pl.ds(k * _LANES,
                                                          _LANES)],)
                                + accs[k + 1:])
                return accs

            accs = tuple(
                jnp.zeros((_LANES,), jnp.float32) for _ in range(nk))
            accs = lax.fori_loop(0, ctx // unroll, jbody, accs)
            for k in range(nk):
                hid_v[r, pl.ds(k * _LANES, _LANES)] = accs[k] * inv
            return 0

        lax.fori_loop(0, bpw, rbody, 0)
        pltpu.sync_copy(hid_v, out_hbm.at[pl.ds(base, bpw)])

    return body(idsT, table)


def _mm_body(wt_ref, h_ref, b_ref, o_ref):
    wt = wt_ref[...].astype(jnp.bfloat16)
    h = h_ref[...].astype(jnp.bfloat16)
    acc = lax.dot_general(wt, h, (((0,), (1,)), ((), ())),
                          preferred_element_type=jnp.float32)
    bias = jnp.transpose(b_ref[...], (1, 0))
    o_ref[...] = acc + bias


def _tc_matmul_t(hidden, Wt, b2, vblk):
    batch, dim = hidden.shape
    vocab = Wt.shape[1]
    grid = (pl.cdiv(vocab, vblk),)
    return pl.pallas_call(
        _mm_body,
        grid=grid,
        in_specs=[
            pl.BlockSpec((dim, vblk), lambda i: (0, i)),
            pl.BlockSpec((batch, dim), lambda i: (0, 0)),
            pl.BlockSpec((1, vblk), lambda i: (0, i)),
        ],
        out_specs=pl.BlockSpec((vblk, batch), lambda i: (i, 0)),
        out_shape=jax.ShapeDtypeStruct((vocab, batch), jnp.float32),
        compiler_params=pltpu.CompilerParams(
            dimension_semantics=("parallel",)),
    )(Wt, hidden, b2)


def kernel(context_ids, emb_table, W, b):
    batch, ctx = context_ids.shape
    vocab, dim = emb_table.shape

    idsT = context_ids.astype(jnp.int32).T
    hidden = _sc_mean_pool(idsT, emb_table, batch, ctx, dim)
    out_t = _tc_matmul_t(hidden, W.T, b.reshape(1, vocab), 2048)
    return out_t.T


# speedup vs baseline: 1.0572x; 1.0572x over previous
import functools

import jax
import jax.numpy as jnp
from jax import lax
from jax.experimental import pallas as pl
from jax.experimental.pallas import tpu as pltpu
from jax.experimental.pallas import tpu_sc as plsc

_NC = 2
_NS = 16
_NW = _NC * _NS
_LANES = 16
_PADW = 128


def _retile_body(tt_ref, o_ref):
    o_ref[:, 0:64] = jnp.transpose(tt_ref[...], (1, 0))


def _tc_retile(tableT, rblk):
    dim, vocab = tableT.shape
    grid = (pl.cdiv(vocab, rblk),)
    return pl.pallas_call(
        _retile_body,
        grid=grid,
        in_specs=[pl.BlockSpec((dim, rblk), lambda i: (0, i))],
        out_specs=pl.BlockSpec((rblk, _PADW), lambda i: (i, 0)),
        out_shape=jax.ShapeDtypeStruct((vocab, _PADW), jnp.float32),
        compiler_params=pltpu.CompilerParams(
            dimension_semantics=("parallel",)),
    )(tableT)


def _sc_mean_pool(ids, table_p, batch, ctx, dim):
    bpw = batch // _NW
    inv = 1.0 / ctx
    nk = dim // _LANES
    unroll = 5
    group = 8
    ngrp = bpw // group
    slots = 2
    cpad = ((ctx + 7) // 8) * 8

    mesh = plsc.VectorSubcoreMesh(core_axis_name="c", subcore_axis_name="s",
                                  num_cores=_NC, num_subcores=_NS)

    @functools.partial(
        pl.kernel,
        out_type=jax.ShapeDtypeStruct((batch, dim), jnp.float32),
        mesh=mesh,
        scratch_types=[
            pltpu.VMEM((bpw, ctx), jnp.int32),
            pltpu.VMEM((slots, group * cpad, _PADW), jnp.float32),
            pltpu.VMEM((bpw, dim), jnp.float32),
            pltpu.SemaphoreType.DMA,
        ],
    )
    def body(ids_hbm, table_hbm, out_hbm, idx_v, rows_v, hid_v, sem):
        wid = lax.axis_index("s") * _NC + lax.axis_index("c")
        base = wid * bpw
        pltpu.sync_copy(ids_hbm.at[pl.ds(base, bpw), :], idx_v)

        def fire(g):
            slot = g % slots
            return [
                pltpu.async_copy(
                    table_hbm.at[idx_v.at[g * group + t]],
                    rows_v.at[slot].at[pl.ds(t * cpad, ctx)],
                    sem,
                )
                for t in range(group)
            ]

        pend = {g: fire(g) for g in range(min(slots, ngrp))}
        for g in range(ngrp):
            for cp in pend.pop(g):
                cp.wait()
            slot = g % slots
            for t in range(group):

                def jbody(j, accs, t=t, slot=slot):
                    for u in range(unroll):
                        row = t * cpad + j * unroll + u
                        for k in range(nk):
                            accs = (accs[:k]
                                    + (accs[k] + rows_v[slot, row,
                                                        pl.ds(k * _LANES,
                                                              _LANES)],)
                                    + accs[k + 1:])
                    return accs

                accs = tuple(
                    jnp.zeros((_LANES,), jnp.float32) for _ in range(nk))
                accs = lax.fori_loop(0, ctx // unroll, jbody, accs)
                for k in range(nk):
                    hid_v[g * group + t,
                          pl.ds(k * _LANES, _LANES)] = accs[k] * inv
            if g + slots < ngrp:
                pend[g + slots] = fire(g + slots)
        pltpu.sync_copy(hid_v, out_hbm.at[pl.ds(base, bpw)])

    return body(ids, table_p)


def _mm_body(wt_ref, h_ref, b_ref, o_ref):
    wt = wt_ref[...].astype(jnp.bfloat16)
    h = h_ref[...].astype(jnp.bfloat16)
    acc = lax.dot_general(wt, h, (((0,), (1,)), ((), ())),
                          preferred_element_type=jnp.float32)
    bias = jnp.transpose(b_ref[...], (1, 0))
    o_ref[...] = acc + bias


def _tc_matmul_t(hidden, Wt, b2, vblk):
    batch, dim = hidden.shape
    vocab = Wt.shape[1]
    grid = (pl.cdiv(vocab, vblk),)
    return pl.pallas_call(
        _mm_body,
        grid=grid,
        in_specs=[
            pl.BlockSpec((dim, vblk), lambda i: (0, i)),
            pl.BlockSpec((batch, dim), lambda i: (0, 0)),
            pl.BlockSpec((1, vblk), lambda i: (0, i)),
        ],
        out_specs=pl.BlockSpec((vblk, batch), lambda i: (i, 0)),
        out_shape=jax.ShapeDtypeStruct((vocab, batch), jnp.float32),
        compiler_params=pltpu.CompilerParams(
            dimension_semantics=("parallel",)),
    )(Wt, hidden, b2)


def kernel(context_ids, emb_table, W, b):
    batch, ctx = context_ids.shape
    vocab, dim = emb_table.shape

    ids = context_ids.astype(jnp.int32)
    table_p = _tc_retile(emb_table.T, 2048)
    hidden = _sc_mean_pool(ids, table_p, batch, ctx, dim)
    out_t = _tc_matmul_t(hidden, W.T, b.reshape(1, vocab), 2048)
    return out_t.T


# speedup vs baseline: 1.1207x; 1.0601x over previous
import functools

import jax
import jax.numpy as jnp
from jax import lax
from jax.experimental import pallas as pl
from jax.experimental.pallas import tpu as pltpu
from jax.experimental.pallas import tpu_sc as plsc

_NC = 2
_NS = 16
_NW = _NC * _NS
_LANES = 16
_PADW = 128


def _retile_body(tt_ref, o_ref):
    dim = tt_ref.shape[0]
    o_ref[:, 0:dim] = jnp.transpose(tt_ref[...], (1, 0))


def _tc_retile(tableT, rblk):
    dim, vocab = tableT.shape
    grid = (pl.cdiv(vocab, rblk),)
    return pl.pallas_call(
        _retile_body,
        grid=grid,
        in_specs=[pl.BlockSpec((dim, rblk), lambda i: (0, i))],
        out_specs=pl.BlockSpec((rblk, _PADW), lambda i: (i, 0)),
        out_shape=jax.ShapeDtypeStruct((vocab, _PADW), jnp.float32),
        compiler_params=pltpu.CompilerParams(
            dimension_semantics=("parallel",)),
    )(tableT)


def _sc_mean_pool(ids, table_p, batch, ctx, dim):
    bpw = batch // _NW
    inv = 1.0 / ctx
    nk = dim // _LANES
    unroll = 5
    group = 8
    ngrp = bpw // group
    slots = 2
    cpad = ((ctx + 7) // 8) * 8

    mesh = plsc.VectorSubcoreMesh(core_axis_name="c", subcore_axis_name="s",
                                  num_cores=_NC, num_subcores=_NS)

    @functools.partial(
        pl.kernel,
        out_type=jax.ShapeDtypeStruct((batch, dim), jnp.float32),
        mesh=mesh,
        scratch_types=[
            pltpu.VMEM((bpw, ctx), jnp.int32),
            pltpu.VMEM((slots, group * cpad, _PADW), jnp.float32),
            pltpu.VMEM((bpw, dim), jnp.float32),
            pltpu.SemaphoreType.DMA,
        ],
    )
    def body(ids_hbm, table_hbm, out_hbm, idx_v, rows_v, hid_v, sem):
        wid = lax.axis_index("s") * _NC + lax.axis_index("c")
        base = wid * bpw
        pltpu.sync_copy(ids_hbm.at[pl.ds(base, bpw), :], idx_v)

        def fire(g):
            slot = g % slots
            return [
                pltpu.async_copy(
                    table_hbm.at[idx_v.at[g * group + t]],
                    rows_v.at[slot].at[pl.ds(t * cpad, ctx)],
                    sem,
                )
                for t in range(group)
            ]

        pend = {g: fire(g) for g in range(min(slots, ngrp))}
        for g in range(ngrp):
            for cp in pend.pop(g):
                cp.wait()
            slot = g % slots
            for t in range(group):

                def jbody(j, accs, t=t, slot=slot):
                    for u in range(unroll):
                        row = t * cpad + j * unroll + u
                        for k in range(nk):
                            accs = (accs[:k]
                                    + (accs[k] + rows_v[slot, row,
                                                        pl.ds(k * _LANES,
                                                              _LANES)],)
                                    + accs[k + 1:])
                    return accs

                accs = tuple(
                    jnp.zeros((_LANES,), jnp.float32) for _ in range(nk))
                accs = lax.fori_loop(0, ctx // unroll, jbody, accs)
                for k in range(nk):
                    hid_v[g * group + t,
                          pl.ds(k * _LANES, _LANES)] = accs[k] * inv
            if g + slots < ngrp:
                pend[g + slots] = fire(g + slots)
        pltpu.sync_copy(hid_v, out_hbm.at[pl.ds(base, bpw)])

    return body(ids, table_p)


def _mm_body(wt_ref, h_ref, b_ref, o_ref):
    wt = wt_ref[...].astype(jnp.bfloat16)
    h = h_ref[...].astype(jnp.bfloat16)
    acc = lax.dot_general(wt, h, (((0,), (1,)), ((), ())),
                          preferred_element_type=jnp.float32)
    bias = jnp.transpose(b_ref[...], (1, 0))
    o_ref[...] = acc + bias


def _tc_matmul_t(hidden, Wt, b2, vblk):
    batch, dim = hidden.shape
    vocab = Wt.shape[1]
    grid = (pl.cdiv(vocab, vblk),)
    return pl.pallas_call(
        _mm_body,
        grid=grid,
        in_specs=[
            pl.BlockSpec((dim, vblk), lambda i: (0, i)),
            pl.BlockSpec((batch, dim), lambda i: (0, 0)),
            pl.BlockSpec((1, vblk), lambda i: (0, i)),
        ],
        out_specs=pl.BlockSpec((vblk, batch), lambda i: (i, 0)),
        out_shape=jax.ShapeDtypeStruct((vocab, batch), jnp.float32),
        compiler_params=pltpu.CompilerParams(
            dimension_semantics=("parallel",)),
    )(Wt, hidden, b2)


def kernel(context_ids, emb_table, W, b):
    batch, ctx = context_ids.shape
    vocab, dim = emb_table.shape

    ids = context_ids.astype(jnp.int32)
    table_p = _tc_retile(emb_table.T, 4096)
    hidden = _sc_mean_pool(ids, table_p, batch, ctx, dim)
    out_t = _tc_matmul_t(hidden, W.T, b.reshape(1, vocab), 2048)
    return out_t.T


# speedup vs baseline: 1.1668x; 1.0411x over previous
import functools

import jax
import jax.numpy as jnp
from jax import lax
from jax.experimental import pallas as pl
from jax.experimental.pallas import tpu as pltpu
from jax.experimental.pallas import tpu_sc as plsc

_NC = 2
_NS = 16
_NW = _NC * _NS
_LANES = 16
_PADW = 128


def _retile_body(tt_ref, o_ref):
    dim = tt_ref.shape[0]
    o_ref[:, 0:dim] = jnp.transpose(tt_ref[...], (1, 0))


def _tc_retile(tableT, rblk):
    dim, vocab = tableT.shape
    grid = (pl.cdiv(vocab, rblk),)
    return pl.pallas_call(
        _retile_body,
        grid=grid,
        in_specs=[pl.BlockSpec((dim, rblk), lambda i: (0, i))],
        out_specs=pl.BlockSpec((rblk, _PADW), lambda i: (i, 0)),
        out_shape=jax.ShapeDtypeStruct((vocab, _PADW), jnp.float32),
        compiler_params=pltpu.CompilerParams(
            dimension_semantics=("parallel",)),
    )(tableT)


def _sc_mean_pool(ids, table_p, batch, ctx, dim):
    bpw = batch // _NW
    inv = 1.0 / ctx
    nk = dim // _LANES
    unroll = 5
    group = 8
    ngrp = bpw // group
    slots = 2
    cpad = ((ctx + 7) // 8) * 8

    mesh = plsc.VectorSubcoreMesh(core_axis_name="c", subcore_axis_name="s",
                                  num_cores=_NC, num_subcores=_NS)

    @functools.partial(
        pl.kernel,
        out_type=jax.ShapeDtypeStruct((batch, dim), jnp.float32),
        mesh=mesh,
        scratch_types=[
            pltpu.VMEM((bpw, ctx), jnp.int32),
            pltpu.VMEM((slots, group * cpad, _PADW), jnp.float32),
            pltpu.VMEM((bpw, dim), jnp.float32),
            pltpu.SemaphoreType.DMA,
        ],
    )
    def body(ids_hbm, table_hbm, out_hbm, idx_v, rows_v, hid_v, sem):
        wid = lax.axis_index("s") * _NC + lax.axis_index("c")
        base = wid * bpw
        pltpu.sync_copy(ids_hbm.at[pl.ds(base, bpw), :], idx_v)

        def fire(g):
            slot = g % slots
            return [
                pltpu.async_copy(
                    table_hbm.at[idx_v.at[g * group + t]],
                    rows_v.at[slot].at[pl.ds(t * cpad, ctx)],
                    sem,
                )
                for t in range(group)
            ]

        pend = {g: fire(g) for g in range(min(slots, ngrp))}
        for g in range(ngrp):
            for cp in pend.pop(g):
                cp.wait()
            slot = g % slots
            for t in range(group):

                def jbody(j, accs, t=t, slot=slot):
                    for u in range(unroll):
                        row = t * cpad + j * unroll + u
                        for k in range(nk):
                            accs = (accs[:k]
                                    + (accs[k] + rows_v[slot, row,
                                                        pl.ds(k * _LANES,
                                                              _LANES)],)
                                    + accs[k + 1:])
                    return accs

                accs = tuple(
                    jnp.zeros((_LANES,), jnp.float32) for _ in range(nk))
                accs = lax.fori_loop(0, ctx // unroll, jbody, accs)
                for k in range(nk):
                    hid_v[g * group + t,
                          pl.ds(k * _LANES, _LANES)] = accs[k] * inv
            if g + slots < ngrp:
                pend[g + slots] = fire(g + slots)
        pltpu.sync_copy(hid_v, out_hbm.at[pl.ds(base, bpw)])

    return body(ids, table_p)


def _mm_body(wt_ref, h_ref, b_ref, o_ref):
    wt = wt_ref[...].astype(jnp.bfloat16)
    h = h_ref[...].astype(jnp.bfloat16)
    acc = lax.dot_general(wt, h, (((0,), (1,)), ((), ())),
                          preferred_element_type=jnp.float32)
    bias = jnp.transpose(b_ref[...], (1, 0))
    o_ref[...] = acc + bias


def _tc_matmul_t(hidden, Wt, b2, vblk):
    batch, dim = hidden.shape
    vocab = Wt.shape[1]
    grid = (pl.cdiv(vocab, vblk),)
    return pl.pallas_call(
        _mm_body,
        grid=grid,
        in_specs=[
            pl.BlockSpec((dim, vblk), lambda i: (0, i)),
            pl.BlockSpec((batch, dim), lambda i: (0, 0)),
            pl.BlockSpec((1, vblk), lambda i: (0, i)),
        ],
        out_specs=pl.BlockSpec((vblk, batch), lambda i: (i, 0)),
        out_shape=jax.ShapeDtypeStruct((vocab, batch), jnp.float32),
        compiler_params=pltpu.CompilerParams(
            dimension_semantics=("parallel",)),
    )(Wt, hidden, b2)


def kernel(context_ids, emb_table, W, b):
    batch, ctx = context_ids.shape
    vocab, dim = emb_table.shape

    ids = context_ids.astype(jnp.int32)
    table_p = _tc_retile(emb_table.T, 12544)
    hidden = _sc_mean_pool(ids, table_p, batch, ctx, dim)
    out_t = _tc_matmul_t(hidden, W.T, b.reshape(1, vocab), 2048)
    return out_t.T


# speedup vs baseline: 1.1877x; 1.0179x over previous
import functools

import jax
import jax.numpy as jnp
from jax import lax
from jax.experimental import pallas as pl
from jax.experimental.pallas import tpu as pltpu
from jax.experimental.pallas import tpu_sc as plsc

_NC = 2
_NS = 16
_NW = _NC * _NS
_LANES = 16
_PADW = 128


def _retile_body(tt_ref, o_ref):
    dim = tt_ref.shape[0]
    o_ref[:, 0:dim] = jnp.transpose(tt_ref[...], (1, 0))


def _tc_retile(tableT, rblk):
    dim, vocab = tableT.shape
    grid = (pl.cdiv(vocab, rblk),)
    return pl.pallas_call(
        _retile_body,
        grid=grid,
        in_specs=[pl.BlockSpec((dim, rblk), lambda i: (0, i))],
        out_specs=pl.BlockSpec((rblk, _PADW), lambda i: (i, 0)),
        out_shape=jax.ShapeDtypeStruct((vocab, _PADW), jnp.float32),
        compiler_params=pltpu.CompilerParams(
            dimension_semantics=("parallel",)),
    )(tableT)


def _sc_mean_pool(ids, table_p, batch, ctx, dim):
    bpw = batch // _NW
    inv = 1.0 / ctx
    nk = dim // _LANES
    unroll = 5
    group = 8
    ngrp = bpw // group
    slots = 2
    cpad = ((ctx + 7) // 8) * 8

    mesh = plsc.VectorSubcoreMesh(core_axis_name="c", subcore_axis_name="s",
                                  num_cores=_NC, num_subcores=_NS)

    @functools.partial(
        pl.kernel,
        out_type=jax.ShapeDtypeStruct((batch, dim), jnp.float32),
        mesh=mesh,
        scratch_types=[
            pltpu.VMEM((bpw, ctx), jnp.int32),
            pltpu.VMEM((slots, group * cpad, _PADW), jnp.float32),
            pltpu.VMEM((bpw, dim), jnp.float32),
            pltpu.SemaphoreType.DMA,
        ],
    )
    def body(ids_hbm, table_hbm, out_hbm, idx_v, rows_v, hid_v, sem):
        wid = lax.axis_index("s") * _NC + lax.axis_index("c")
        base = wid * bpw
        pltpu.sync_copy(ids_hbm.at[pl.ds(base, bpw), :], idx_v)

        def fire(g):
            slot = g % slots
            return [
                pltpu.async_copy(
                    table_hbm.at[idx_v.at[g * group + t]],
                    rows_v.at[slot].at[pl.ds(t * cpad, ctx)],
                    sem,
                )
                for t in range(group)
            ]

        pend = {g: fire(g) for g in range(min(slots, ngrp))}
        for g in range(ngrp):
            for cp in pend.pop(g):
                cp.wait()
            slot = g % slots
            for t in range(group):

                def jbody(j, accs, t=t, slot=slot):
                    for u in range(unroll):
                        row = t * cpad + j * unroll + u
                        for k in range(nk):
                            accs = (accs[:k]
                                    + (accs[k] + rows_v[slot, row,
                                                        pl.ds(k * _LANES,
                                                              _LANES)],)
                                    + accs[k + 1:])
                    return accs

                accs = tuple(
                    jnp.zeros((_LANES,), jnp.float32) for _ in range(nk))
                accs = lax.fori_loop(0, ctx // unroll, jbody, accs)
                for k in range(nk):
                    hid_v[g * group + t,
                          pl.ds(k * _LANES, _LANES)] = accs[k] * inv
            if g + slots < ngrp:
                pend[g + slots] = fire(g + slots)
        pltpu.sync_copy(hid_v, out_hbm.at[pl.ds(base, bpw)])

    return body(ids, table_p)


def _mm_body(wt_ref, h_ref, b_ref, o_ref):
    wt = wt_ref[...].astype(jnp.bfloat16)
    h = h_ref[...].astype(jnp.bfloat16)
    acc = lax.dot_general(wt, h, (((0,), (1,)), ((), ())),
                          preferred_element_type=jnp.float32)
    bias = jnp.transpose(b_ref[...], (1, 0))
    o_ref[...] = acc + bias


def _tc_matmul_t(hidden, Wt, b2, vblk):
    batch, dim = hidden.shape
    vocab = Wt.shape[1]
    grid = (pl.cdiv(vocab, vblk),)
    return pl.pallas_call(
        _mm_body,
        grid=grid,
        in_specs=[
            pl.BlockSpec((dim, vblk), lambda i: (0, i)),
            pl.BlockSpec((batch, dim), lambda i: (0, 0)),
            pl.BlockSpec((1, vblk), lambda i: (0, i)),
        ],
        out_specs=pl.BlockSpec((vblk, batch), lambda i: (i, 0)),
        out_shape=jax.ShapeDtypeStruct((vocab, batch), jnp.float32),
        compiler_params=pltpu.CompilerParams(
            dimension_semantics=("parallel",)),
    )(Wt, hidden, b2)


def kernel(context_ids, emb_table, W, b):
    batch, ctx = context_ids.shape
    vocab, dim = emb_table.shape

    ids = context_ids.astype(jnp.int32)
    table_p = _tc_retile(emb_table.T, 25088)
    hidden = _sc_mean_pool(ids, table_p, batch, ctx, dim)
    out_t = _tc_matmul_t(hidden, W.T, b.reshape(1, vocab), 4096)
    return out_t.T
